# Initial kernel scaffold; baseline (speedup 1.0000x reference)
import jax
import jax.numpy as jnp
from jax.experimental import pallas as pl


def kernel(t):
    raise NotImplementedError("write your pallas kernel here")


# speedup vs baseline: 327.3378x; 327.3378x over previous
import functools

import jax
import jax.numpy as jnp
from jax import lax
from jax.experimental import pallas as pl
from jax.experimental.pallas import tpu as pltpu
from jax.experimental.pallas import tpu_sc as plsc

N_ROWS = 100000
N_COLS = 256
N_IDX = N_ROWS * N_COLS
NC, NS = 2, 16
NW = NC * NS
PER_W = N_IDX // NW
CHUNK = 10000
NCH = PER_W // CHUNK
ROW_BLK = 1000
NG = N_ROWS // ROW_BLK


def _sc_mark_hits(idx_hbm, out_hbm, mask_v, buf0, buf1, sem0, sem1):
    c = lax.axis_index("c")
    s = lax.axis_index("s")
    wid = s * NC + c
    base = wid * PER_W

    zeros16 = jnp.zeros((16,), jnp.float32)
    ones16 = jnp.ones((16,), jnp.float32)

    def zero_body(i, _):
        mask_v[pl.ds(i * 16, 16)] = zeros16
        return _

    lax.fori_loop(0, N_ROWS // 16, zero_body, None, unroll=4)

    bufs = (buf0, buf1)
    sems = (sem0, sem1)
    descs = [None] * NCH
    for ci in range(2):
        descs[ci] = pltpu.async_copy(
            idx_hbm.at[pl.ds(base + ci * CHUNK, CHUNK)], bufs[ci], sems[ci])

    for ci in range(NCH):
        buf = bufs[ci % 2]
        descs[ci].wait()

        def scat_body(j, _, buf=buf):
            iv = buf[pl.ds(j * 16, 16)]
            plsc.store_scatter(mask_v, [iv], ones16)
            return _

        lax.fori_loop(0, CHUNK // 16, scat_body, None, unroll=8)
        if ci + 2 < NCH:
            descs[ci + 2] = pltpu.async_copy(
                idx_hbm.at[pl.ds(base + (ci + 2) * CHUNK, CHUNK)],
                buf, sems[ci % 2])

    pltpu.sync_copy(mask_v, out_hbm.at[wid])


_sc_mark_hits_call = functools.partial(
    pl.kernel,
    mesh=plsc.VectorSubcoreMesh(core_axis_name="c", subcore_axis_name="s"),
    out_type=jax.ShapeDtypeStruct((NW, N_ROWS), jnp.float32),
    scratch_types=[
        pltpu.VMEM((N_ROWS,), jnp.float32),
        pltpu.VMEM((CHUNK,), jnp.int32),
        pltpu.VMEM((CHUNK,), jnp.int32),
        pltpu.SemaphoreType.DMA,
        pltpu.SemaphoreType.DMA,
    ],
    compiler_params=pltpu.CompilerParams(needs_layout_passes=False),
)(_sc_mark_hits)


def _tc_masked_sum(t_ref, m_ref, sum_ref, cnt_ref):
    i = pl.program_id(0)

    @pl.when(i == 0)
    def _():
        sum_ref[...] = jnp.zeros_like(sum_ref)
        cnt_ref[...] = jnp.zeros_like(cnt_ref)

    m2 = m_ref[...].reshape(NW, ROW_BLK)
    hitc = jnp.sum(m2, axis=0, keepdims=True)
    missf = jnp.where(hitc > 0.0, 0.0, 1.0)
    rows = jnp.sum(t_ref[...], axis=1, keepdims=True)
    part = jnp.dot(missf, rows, preferred_element_type=jnp.float32)
    sum_ref[...] += part
    cnt_ref[...] += jnp.sum(missf, axis=1, keepdims=True)


def _tc_masked_sum_call(t, hits):
    return pl.pallas_call(
        _tc_masked_sum,
        grid=(NG,),
        in_specs=[
            pl.BlockSpec((ROW_BLK, N_COLS), lambda i: (i, 0)),
            pl.BlockSpec((NW, 1, 1, ROW_BLK), lambda i: (0, i, 0, 0)),
        ],
        out_specs=[
            pl.BlockSpec((1, 1), lambda i: (0, 0)),
            pl.BlockSpec((1, 1), lambda i: (0, 0)),
        ],
        out_shape=[
            jax.ShapeDtypeStruct((1, 1), jnp.float32),
            jax.ShapeDtypeStruct((1, 1), jnp.float32),
        ],
    )(t, hits)


def kernel(t):
    assert t.shape == (N_ROWS, N_COLS)
    k1, k2 = jax.random.split(jax.random.key(1))
    index = jax.random.randint(k1, t.shape, 0, t.shape[0], dtype=jnp.int32)
    val = jax.random.normal(k2, (1,), dtype=t.dtype)

    hits = _sc_mark_hits_call(index.reshape(-1))
    hits4d = hits.reshape(NW, NG, 1, ROW_BLK)
    s_missed, n_missed = _tc_masked_sum_call(t, hits4d)
    s_missed = s_missed[0, 0]
    n_missed = n_missed[0, 0]
    n_hit = N_ROWS - n_missed
    return (s_missed + n_hit * (N_COLS * val[0])) / N_IDX


# speedup vs baseline: 484.7474x; 1.4809x over previous
import functools

import jax
import jax.numpy as jnp
from jax import lax
from jax.experimental import pallas as pl
from jax.experimental.pallas import tpu as pltpu
from jax.experimental.pallas import tpu_sc as plsc

N_ROWS = 100000
N_COLS = 256
N_IDX = N_ROWS * N_COLS
NC, NS = 2, 16
NW = NC * NS
PER_W = N_IDX // NW
CHUNK = 10000
NCH = PER_W // CHUNK
ROW_BLK = 1000
NG = N_ROWS // ROW_BLK


def _sc_mark_hits(idx_hbm, out_hbm, mask_v, buf0, buf1, sem0, sem1):
    c = lax.axis_index("c")
    s = lax.axis_index("s")
    wid = s * NC + c
    base = wid * PER_W

    zeros16 = jnp.zeros((16,), jnp.float32)
    ones16 = jnp.ones((16,), jnp.float32)

    @plsc.parallel_loop(0, N_ROWS // 16, 1, unroll=8)
    def zero_body(i):
        mask_v[pl.ds(i * 16, 16)] = zeros16

    bufs = (buf0, buf1)
    sems = (sem0, sem1)
    descs = [None] * NCH
    for ci in range(2):
        descs[ci] = pltpu.async_copy(
            idx_hbm.at[pl.ds(base + ci * CHUNK, CHUNK)], bufs[ci], sems[ci])

    for ci in range(NCH):
        buf = bufs[ci % 2]
        descs[ci].wait()

        @plsc.parallel_loop(0, CHUNK // 16, 1, unroll=8)
        def scat_body(j, buf=buf):
            iv = buf[pl.ds(j * 16, 16)]
            plsc.store_scatter(mask_v, [iv], ones16)
        if ci + 2 < NCH:
            descs[ci + 2] = pltpu.async_copy(
                idx_hbm.at[pl.ds(base + (ci + 2) * CHUNK, CHUNK)],
                buf, sems[ci % 2])

    pltpu.sync_copy(mask_v, out_hbm.at[wid])


_sc_mark_hits_call = functools.partial(
    pl.kernel,
    mesh=plsc.VectorSubcoreMesh(core_axis_name="c", subcore_axis_name="s"),
    out_type=jax.ShapeDtypeStruct((NW, N_ROWS), jnp.float32),
    scratch_types=[
        pltpu.VMEM((N_ROWS,), jnp.float32),
        pltpu.VMEM((CHUNK,), jnp.int32),
        pltpu.VMEM((CHUNK,), jnp.int32),
        pltpu.SemaphoreType.DMA,
        pltpu.SemaphoreType.DMA,
    ],
    compiler_params=pltpu.CompilerParams(needs_layout_passes=False),
)(_sc_mark_hits)


def _tc_masked_sum(t_ref, m_ref, sum_ref, cnt_ref):
    i = pl.program_id(0)

    @pl.when(i == 0)
    def _():
        sum_ref[...] = jnp.zeros_like(sum_ref)
        cnt_ref[...] = jnp.zeros_like(cnt_ref)

    m2 = m_ref[...].reshape(NW, ROW_BLK)
    hitc = jnp.sum(m2, axis=0, keepdims=True)
    missf = jnp.where(hitc > 0.0, 0.0, 1.0)
    rows = jnp.sum(t_ref[...], axis=1, keepdims=True)
    part = jnp.dot(missf, rows, preferred_element_type=jnp.float32)
    sum_ref[...] += part
    cnt_ref[...] += jnp.sum(missf, axis=1, keepdims=True)


def _tc_masked_sum_call(t, hits):
    return pl.pallas_call(
        _tc_masked_sum,
        grid=(NG,),
        in_specs=[
            pl.BlockSpec((ROW_BLK, N_COLS), lambda i: (i, 0)),
            pl.BlockSpec((NW, 1, 1, ROW_BLK), lambda i: (0, i, 0, 0)),
        ],
        out_specs=[
            pl.BlockSpec((1, 1), lambda i: (0, 0)),
            pl.BlockSpec((1, 1), lambda i: (0, 0)),
        ],
        out_shape=[
            jax.ShapeDtypeStruct((1, 1), jnp.float32),
            jax.ShapeDtypeStruct((1, 1), jnp.float32),
        ],
    )(t, hits)


def kernel(t):
    assert t.shape == (N_ROWS, N_COLS)
    k1, k2 = jax.random.split(jax.random.key(1))
    index = jax.random.randint(k1, (N_IDX,), 0, t.shape[0], dtype=jnp.int32)
    val = jax.random.normal(k2, (1,), dtype=t.dtype)

    hits = _sc_mark_hits_call(index)
    hits4d = hits.reshape(NW, NG, 1, ROW_BLK)
    s_missed, n_missed = _tc_masked_sum_call(t, hits4d)
    s_missed = s_missed[0, 0]
    n_missed = n_missed[0, 0]
    n_hit = N_ROWS - n_missed
    return (s_missed + n_hit * (N_COLS * val[0])) / N_IDX


# speedup vs baseline: 547.8837x; 1.1302x over previous
import functools

import jax
import jax.numpy as jnp
from jax import lax
from jax.experimental import pallas as pl
from jax.experimental.pallas import tpu as pltpu
from jax.experimental.pallas import tpu_sc as plsc

N_ROWS = 100000
N_COLS = 256
N_IDX = N_ROWS * N_COLS
NC, NS = 2, 16
NW = NC * NS
PER_W = N_IDX // NW
CHUNK = 10000
NCH = PER_W // CHUNK
M_PAD = 100352
NPASS = 14
HALF = M_PAD // NPASS
HSLICE = HALF // NS
ROW_BLK = 1000
NG = N_ROWS // ROW_BLK


def _sc_mark_hits(idx_hbm, out_hbm, mask_v, buf0, buf1, shared, sem0, sem1):
    c = lax.axis_index("c")
    s = lax.axis_index("s")
    wid = s * NC + c
    base = wid * PER_W

    zeros16 = jnp.zeros((16,), jnp.float32)
    ones16 = jnp.ones((16,), jnp.float32)

    @plsc.parallel_loop(0, M_PAD // 16, 1, unroll=8)
    def zero_body(i):
        mask_v[pl.ds(i * 16, 16)] = zeros16

    bufs = (buf0, buf1)
    sems = (sem0, sem1)
    descs = [None] * NCH
    for ci in range(2):
        descs[ci] = pltpu.async_copy(
            idx_hbm.at[pl.ds(base + ci * CHUNK, CHUNK)], bufs[ci], sems[ci])

    for ci in range(NCH):
        buf = bufs[ci % 2]
        descs[ci].wait()

        @plsc.parallel_loop(0, CHUNK // 16, 1, unroll=8)
        def scat_body(j, buf=buf):
            iv = buf[pl.ds(j * 16, 16)]
            plsc.store_scatter(mask_v, [iv], ones16)

        if ci + 2 < NCH:
            descs[ci + 2] = pltpu.async_copy(
                idx_hbm.at[pl.ds(base + (ci + 2) * CHUNK, CHUNK)],
                buf, sems[ci % 2])

    hoff = s * HSLICE

    def pass_body(p, carry):
        plsc.subcore_barrier()
        pltpu.sync_copy(mask_v.at[pl.ds(p * HALF, HALF)],
                        shared.at[pl.ds(s * HALF, HALF)])
        plsc.subcore_barrier()

        pltpu.sync_copy(shared.at[pl.ds(hoff, HSLICE)],
                        mask_v.at[pl.ds(0, HSLICE)])

        def slot_body(j, carry2):
            pltpu.sync_copy(shared.at[pl.ds(j * HALF + hoff, HSLICE)],
                            mask_v.at[pl.ds(HSLICE, HSLICE)])

            @plsc.parallel_loop(0, HSLICE // 16, 1, unroll=4)
            def add_body(i):
                a = mask_v[pl.ds(i * 16, 16)]
                b = mask_v[pl.ds(HSLICE + i * 16, 16)]
                mask_v[pl.ds(i * 16, 16)] = a + b

            return carry2

        lax.fori_loop(1, NS, slot_body, None)
        pltpu.sync_copy(mask_v.at[pl.ds(0, HSLICE)],
                        out_hbm.at[pl.ds(c * M_PAD + p * HALF + hoff, HSLICE)])
        return carry

    lax.fori_loop(0, NPASS, pass_body, None)


_sc_mark_hits_call = functools.partial(
    pl.kernel,
    mesh=plsc.VectorSubcoreMesh(core_axis_name="c", subcore_axis_name="s"),
    out_type=jax.ShapeDtypeStruct((NC * M_PAD,), jnp.float32),
    scratch_types=[
        pltpu.VMEM((M_PAD,), jnp.float32),
        pltpu.VMEM((CHUNK,), jnp.int32),
        pltpu.VMEM((CHUNK,), jnp.int32),
        pltpu.VMEM_SHARED((NS * HALF,), jnp.float32),
        pltpu.SemaphoreType.DMA,
        pltpu.SemaphoreType.DMA,
    ],
    compiler_params=pltpu.CompilerParams(needs_layout_passes=False),
)(_sc_mark_hits)


def _tc_rowsum(t_ref, e_ref, out_ref):
    rows = jnp.sum(t_ref[...], axis=1, keepdims=True)
    r_t = lax.dot_general(rows, e_ref[...], (((0,), (0,)), ((), ())),
                          preferred_element_type=jnp.float32)
    out_ref[...] = r_t.reshape(1, 1, ROW_BLK)


def _tc_rowsum_call(t, eye):
    return pl.pallas_call(
        _tc_rowsum,
        grid=(NG,),
        in_specs=[
            pl.BlockSpec((ROW_BLK, N_COLS), lambda i: (i, 0)),
            pl.BlockSpec((ROW_BLK, ROW_BLK), lambda i: (0, 0)),
        ],
        out_specs=pl.BlockSpec((1, 1, ROW_BLK), lambda i: (i, 0, 0)),
        out_shape=jax.ShapeDtypeStruct((NG, 1, ROW_BLK), jnp.float32),
    )(t, eye)


def _tc_final(hm_ref, r_ref, v_ref, out_ref):
    m = hm_ref[0] + hm_ref[1]
    r = r_ref[...]
    miss = jnp.where(m > 0.0, 0.0, r)
    nmiss = jnp.where(m > 0.0, 0.0, 1.0)
    s = jnp.sum(miss)
    n = jnp.sum(nmiss)
    v = v_ref[...]
    out_ref[...] = (s + (N_ROWS - n) * float(N_COLS) * v) / float(N_IDX)


def _tc_final_call(hm, rsum, val2d):
    return pl.pallas_call(
        _tc_final,
        grid=(1,),
        in_specs=[
            pl.BlockSpec((NC, NG, 1, ROW_BLK), lambda i: (0, 0, 0, 0)),
            pl.BlockSpec((NG, 1, ROW_BLK), lambda i: (0, 0, 0)),
            pl.BlockSpec((1, 1), lambda i: (0, 0)),
        ],
        out_specs=pl.BlockSpec((1, 1), lambda i: (0, 0)),
        out_shape=jax.ShapeDtypeStruct((1, 1), jnp.float32),
    )(hm, rsum, val2d)


def kernel(t):
    assert t.shape == (N_ROWS, N_COLS)
    k1, k2 = jax.random.split(jax.random.key(1))
    index = jax.random.randint(k1, (N_IDX,), 0, t.shape[0], dtype=jnp.int32)
    val = jax.random.normal(k2, (1,), dtype=t.dtype)

    hits = _sc_mark_hits_call(index)
    hm = hits.reshape(NC, M_PAD)[:, :N_ROWS].reshape(NC, NG, 1, ROW_BLK)
    eye = jnp.eye(ROW_BLK, dtype=jnp.float32)
    rsum = _tc_rowsum_call(t, eye)
    out = _tc_final_call(hm, rsum, val.reshape(1, 1))
    return out[0, 0]
